# Initial kernel scaffold; baseline (speedup 1.0000x reference)
import functools

import jax
import jax.numpy as jnp
from jax import lax
from jax.experimental import pallas as pl
from jax.experimental.pallas import tpu as pltpu
from jax.experimental.pallas import tpu_sc as plsc

N = 10000
E = 320000
F_IN = 128
H = 128
C = 40

NC = 2
NS = 16
NW = NC * NS
LANES = 16

K = 128
CHUNKS = E // K
CHUNKS_PER_W = (CHUNKS + NW - 1) // NW
ROWS_PER_SUB = N // NS

ROW_BLK = 2000


def _spmm_sc(support, src, dst, ew):
    mesh = plsc.VectorSubcoreMesh(
        core_axis_name="c", subcore_axis_name="s", num_cores=NC, num_subcores=NS
    )

    @functools.partial(
        pl.kernel,
        out_type=jax.ShapeDtypeStruct((NC, N, H), jnp.float32),
        mesh=mesh,
        scratch_types=[
            pltpu.VMEM((K, H), jnp.float32),
            pltpu.VMEM((K,), jnp.int32),
            pltpu.VMEM((K,), jnp.int32),
            pltpu.VMEM((K,), jnp.float32),
            pltpu.VMEM_SHARED((N, H), jnp.float32),
            pltpu.SemaphoreType.DMA,
        ],
    )
    def spmm(sup_hbm, src_hbm, dst_hbm, ew_hbm, out_hbm, rows_v, si_v, di_v,
             ew_v, acc, sem):
        cid = lax.axis_index("c")
        sid = lax.axis_index("s")
        wid = sid * NC + cid

        @pl.loop(0, K)
        def _(r):
            for f in range(H // LANES):
                rows_v[r, pl.ds(f * LANES, LANES)] = jnp.zeros(
                    (LANES,), jnp.float32)

        base = sid * ROWS_PER_SUB
        nfull = ROWS_PER_SUB // K
        rem = ROWS_PER_SUB - nfull * K
        for blk in range(nfull):
            pltpu.sync_copy(rows_v, acc.at[pl.ds(base + blk * K, K)])
        if rem:
            pltpu.sync_copy(rows_v.at[pl.ds(0, rem)],
                            acc.at[pl.ds(base + nfull * K, rem)])
        plsc.subcore_barrier()

        @pl.loop(0, CHUNKS_PER_W)
        def _(i):
            chunk = wid + i * NW

            @pl.when(chunk < CHUNKS)
            def _():
                eb = chunk * K
                pltpu.sync_copy(src_hbm.at[pl.ds(eb, K)], si_v)
                pltpu.sync_copy(dst_hbm.at[pl.ds(eb, K)], di_v)
                pltpu.sync_copy(ew_hbm.at[pl.ds(eb, K)], ew_v)
                pltpu.async_copy(sup_hbm.at[si_v], rows_v, sem).wait()

                @pl.loop(0, K)
                def _(e):
                    widx = jnp.full((LANES,), e, jnp.int32)
                    wv = plsc.load_gather(ew_v, [widx])
                    for f in range(H // LANES):
                        sl = pl.ds(f * LANES, LANES)
                        rows_v[e, sl] = rows_v[e, sl] * wv

                pltpu.sync_copy(rows_v, acc.at[di_v], add=True)

        plsc.subcore_barrier()
        pltpu.sync_copy(acc.at[pl.ds(base, ROWS_PER_SUB)],
                        out_hbm.at[cid, pl.ds(base, ROWS_PER_SUB)])

    return spmm(support, src, dst, ew)


def _mm_first(x, W):
    def body(x_ref, w_ref, o_ref):
        o_ref[...] = jnp.dot(x_ref[...], w_ref[...],
                             preferred_element_type=jnp.float32)

    return pl.pallas_call(
        body,
        grid=(N // ROW_BLK,),
        in_specs=[
            pl.BlockSpec((ROW_BLK, F_IN), lambda i: (i, 0)),
            pl.BlockSpec((F_IN, H), lambda i: (0, 0)),
        ],
        out_specs=pl.BlockSpec((ROW_BLK, H), lambda i: (i, 0)),
        out_shape=jax.ShapeDtypeStruct((N, H), jnp.float32),
    )(x, W)


def _mm_mid(parts, b, W):
    def body(p_ref, b_ref, w_ref, o_ref):
        h = jnp.maximum(p_ref[0] + p_ref[1] + b_ref[...], 0.0)
        o_ref[...] = jnp.dot(h, w_ref[...],
                             preferred_element_type=jnp.float32)

    return pl.pallas_call(
        body,
        grid=(N // ROW_BLK,),
        in_specs=[
            pl.BlockSpec((NC, ROW_BLK, H), lambda i: (0, i, 0)),
            pl.BlockSpec((1, H), lambda i: (0, 0)),
            pl.BlockSpec((H, H), lambda i: (0, 0)),
        ],
        out_specs=pl.BlockSpec((ROW_BLK, H), lambda i: (i, 0)),
        out_shape=jax.ShapeDtypeStruct((N, H), jnp.float32),
    )(parts, b, W)


def _mm_out(parts, b, W, b_out):
    def body(p_ref, b_ref, w_ref, bo_ref, o_ref):
        h = jnp.maximum(p_ref[0] + p_ref[1] + b_ref[...], 0.0)
        logits = jnp.dot(h, w_ref[...],
                         preferred_element_type=jnp.float32) + bo_ref[...]
        m = jnp.max(logits, axis=1, keepdims=True)
        shifted = logits - m
        lse = jnp.log(jnp.sum(jnp.exp(shifted), axis=1, keepdims=True))
        o_ref[...] = shifted - lse

    return pl.pallas_call(
        body,
        grid=(N // ROW_BLK,),
        in_specs=[
            pl.BlockSpec((NC, ROW_BLK, H), lambda i: (0, i, 0)),
            pl.BlockSpec((1, H), lambda i: (0, 0)),
            pl.BlockSpec((H, C), lambda i: (0, 0)),
            pl.BlockSpec((1, C), lambda i: (0, 0)),
        ],
        out_specs=pl.BlockSpec((ROW_BLK, C), lambda i: (i, 0)),
        out_shape=jax.ShapeDtypeStruct((N, C), jnp.float32),
    )(parts, b, W, b_out)


def kernel(x, edge_index, edge_weight, W1, b1, W2, b2, W_out, b_out):
    src = edge_index[0]
    dst = edge_index[1]
    b1r = b1.reshape(1, H)
    b2r = b2.reshape(1, H)
    bor = b_out.reshape(1, C)

    support1 = _mm_first(x, W1)
    parts1 = _spmm_sc(support1, src, dst, edge_weight)
    support2 = _mm_mid(parts1, b1r, W2)
    parts2 = _spmm_sc(support2, src, dst, edge_weight)
    return _mm_out(parts2, b2r, W_out, bor)


# speedup vs baseline: 4.4780x; 4.4780x over previous
import dataclasses
import functools

import jax
import jax.numpy as jnp
from jax import lax
from jax.experimental import pallas as pl
from jax.experimental.pallas import tpu as pltpu
from jax.experimental.pallas import tpu_sc as plsc

N = 10000
E = 320000
F_IN = 128
H = 128
C = 40

NC = 2
NS = 16
NW = NC * NS
LANES = 16

K = 128
CHUNKS = E // K
CHUNKS_PER_W = (CHUNKS + NW - 1) // NW
ROWS_PER_SUB = 624
ROWS_TAIL = N - NS * ROWS_PER_SUB

ROW_BLK = 2000


def _spmm_sc(support, src, dst, ew):
    mesh = plsc.VectorSubcoreMesh(
        core_axis_name="c", subcore_axis_name="s", num_cores=NC, num_subcores=NS
    )
    cp = pltpu.CompilerParams()
    if "needs_layout_passes" in pltpu.CompilerParams.__dataclass_fields__:
        cp = dataclasses.replace(cp, needs_layout_passes=False)

    @functools.partial(
        pl.kernel,
        compiler_params=cp,
        out_type=jax.ShapeDtypeStruct((NC, N, H), jnp.float32),
        mesh=mesh,
        scratch_types=[
            pltpu.VMEM((K, H), jnp.float32),
            pltpu.VMEM((K,), jnp.int32),
            pltpu.VMEM((K,), jnp.int32),
            pltpu.VMEM((K,), jnp.float32),
            pltpu.VMEM_SHARED((N, H), jnp.float32),
            pltpu.SemaphoreType.DMA,
        ],
    )
    def spmm(sup_hbm, src_hbm, dst_hbm, ew_hbm, out_hbm, rows_v, si_v, di_v,
             ew_v, acc, sem):
        cid = lax.axis_index("c")
        sid = lax.axis_index("s")
        wid = sid * NC + cid

        @pl.loop(0, K)
        def _(r):
            for f in range(H // LANES):
                rows_v[r, pl.ds(f * LANES, LANES)] = jnp.zeros(
                    (LANES,), jnp.float32)

        base = sid * ROWS_PER_SUB
        nfull = ROWS_PER_SUB // K
        rem = ROWS_PER_SUB - nfull * K
        for blk in range(nfull):
            pltpu.sync_copy(rows_v, acc.at[pl.ds(base + blk * K, K)])
        if rem:
            pltpu.sync_copy(rows_v.at[pl.ds(0, rem)],
                            acc.at[pl.ds(base + nfull * K, rem)])

        @pl.when(sid == NS - 1)
        def _():
            pltpu.sync_copy(rows_v.at[pl.ds(0, ROWS_TAIL)],
                            acc.at[pl.ds(NS * ROWS_PER_SUB, ROWS_TAIL)])

        plsc.subcore_barrier()

        @pl.loop(0, CHUNKS_PER_W)
        def _(i):
            chunk = wid + i * NW

            @pl.when(chunk < CHUNKS)
            def _():
                eb = chunk * K
                pltpu.sync_copy(src_hbm.at[pl.ds(eb, K)], si_v)
                pltpu.sync_copy(dst_hbm.at[pl.ds(eb, K)], di_v)
                pltpu.sync_copy(ew_hbm.at[pl.ds(eb, K)], ew_v)
                pltpu.async_copy(sup_hbm.at[si_v], rows_v, sem).wait()

                @pl.loop(0, K)
                def _(e):
                    widx = jnp.full((LANES,), e, jnp.int32)
                    wv = plsc.load_gather(ew_v, [widx])
                    for f in range(H // LANES):
                        sl = pl.ds(f * LANES, LANES)
                        rows_v[e, sl] = rows_v[e, sl] * wv

                pltpu.sync_copy(rows_v, acc.at[di_v], add=True)

        plsc.subcore_barrier()
        pltpu.sync_copy(acc.at[pl.ds(base, ROWS_PER_SUB)],
                        out_hbm.at[cid, pl.ds(base, ROWS_PER_SUB)])

        @pl.when(sid == NS - 1)
        def _():
            pltpu.sync_copy(acc.at[pl.ds(NS * ROWS_PER_SUB, ROWS_TAIL)],
                            out_hbm.at[cid, pl.ds(NS * ROWS_PER_SUB,
                                                  ROWS_TAIL)])

    return spmm(support, src, dst, ew)


def _mm_first(x, W):
    def body(x_ref, w_ref, o_ref):
        o_ref[...] = jnp.dot(x_ref[...], w_ref[...],
                             preferred_element_type=jnp.float32)

    return pl.pallas_call(
        body,
        grid=(N // ROW_BLK,),
        in_specs=[
            pl.BlockSpec((ROW_BLK, F_IN), lambda i: (i, 0)),
            pl.BlockSpec((F_IN, H), lambda i: (0, 0)),
        ],
        out_specs=pl.BlockSpec((ROW_BLK, H), lambda i: (i, 0)),
        out_shape=jax.ShapeDtypeStruct((N, H), jnp.float32),
    )(x, W)


def _mm_mid(parts, b, W):
    def body(p_ref, b_ref, w_ref, o_ref):
        h = jnp.maximum(p_ref[0] + p_ref[1] + b_ref[...], 0.0)
        o_ref[...] = jnp.dot(h, w_ref[...],
                             preferred_element_type=jnp.float32)

    return pl.pallas_call(
        body,
        grid=(N // ROW_BLK,),
        in_specs=[
            pl.BlockSpec((NC, ROW_BLK, H), lambda i: (0, i, 0)),
            pl.BlockSpec((1, H), lambda i: (0, 0)),
            pl.BlockSpec((H, H), lambda i: (0, 0)),
        ],
        out_specs=pl.BlockSpec((ROW_BLK, H), lambda i: (i, 0)),
        out_shape=jax.ShapeDtypeStruct((N, H), jnp.float32),
    )(parts, b, W)


def _mm_out(parts, b, W, b_out):
    def body(p_ref, b_ref, w_ref, bo_ref, o_ref):
        h = jnp.maximum(p_ref[0] + p_ref[1] + b_ref[...], 0.0)
        logits = jnp.dot(h, w_ref[...],
                         preferred_element_type=jnp.float32) + bo_ref[...]
        m = jnp.max(logits, axis=1, keepdims=True)
        shifted = logits - m
        lse = jnp.log(jnp.sum(jnp.exp(shifted), axis=1, keepdims=True))
        o_ref[...] = shifted - lse

    return pl.pallas_call(
        body,
        grid=(N // ROW_BLK,),
        in_specs=[
            pl.BlockSpec((NC, ROW_BLK, H), lambda i: (0, i, 0)),
            pl.BlockSpec((1, H), lambda i: (0, 0)),
            pl.BlockSpec((H, C), lambda i: (0, 0)),
            pl.BlockSpec((1, C), lambda i: (0, 0)),
        ],
        out_specs=pl.BlockSpec((ROW_BLK, C), lambda i: (i, 0)),
        out_shape=jax.ShapeDtypeStruct((N, C), jnp.float32),
    )(parts, b, W, b_out)


def kernel(x, edge_index, edge_weight, W1, b1, W2, b2, W_out, b_out):
    src = edge_index[0]
    dst = edge_index[1]
    b1r = b1.reshape(1, H)
    b2r = b2.reshape(1, H)
    bor = b_out.reshape(1, C)

    support1 = _mm_first(x, W1)
    parts1 = _spmm_sc(support1, src, dst, edge_weight)
    support2 = _mm_mid(parts1, b1r, W2)
    parts2 = _spmm_sc(support2, src, dst, edge_weight)
    return _mm_out(parts2, b2r, W_out, bor)
